# baseline (speedup 1.0000x reference)
import functools

import jax
import jax.numpy as jnp
from jax import lax
from jax.experimental import pallas as pl
from jax.experimental.pallas import tpu as pltpu
from jax.experimental.pallas import tpu_sc as plsc

N = 32768
B = 16
C = 96
NC = 2
NS = 16
NW = NC * NS
RPW = N // NW
NVC = C // 16
EPS = 1e-5


CR = 256
CH = RPW // CR


def _sc_body(feat_hbm, cu_hbm, out_hbm, buf0, buf1, cu_v, acc_v,
             sem0, sem1):
    wid = lax.axis_index("s") * NC + lax.axis_index("c")
    r0 = wid * RPW
    bufs = (buf0, buf1)
    sems = (sem0, sem1)

    def start(c):
        return pltpu.async_copy(
            feat_hbm.at[pl.ds(r0 + c * CR, CR)], bufs[c % 2], sems[c % 2])

    cp0 = start(0)
    pltpu.sync_copy(cu_hbm, cu_v)
    cu_a = cu_v[pl.ds(0, 16)]
    cu_b = cu_v[pl.ds(16, 16)]
    cu_s = [cu_a[j] if j < 16 else cu_b[j - 16] for j in range(B + 1)]
    copies = [cp0]
    for c in range(CH):
        if c + 1 < CH:
            copies.append(start(c + 1))
        copies[c].wait()
        buf = bufs[c % 2]
        c0 = r0 + c * CR
        for b in range(B):
            lo = jnp.maximum(cu_s[b], c0)
            hi = jnp.minimum(cu_s[b + 1], c0 + CR)

            def body(i, carry):
                il = i - c0
                return tuple(carry[k] + buf[il, pl.ds(16 * k, 16)]
                             for k in range(NVC))

            init = tuple(acc_v[b, pl.ds(16 * k, 16)] for k in range(NVC)) \
                if c else tuple(jnp.zeros((16,), jnp.float32)
                                for _ in range(NVC))
            accs = lax.fori_loop(lo, hi, body, init)
            for k in range(NVC):
                acc_v[b, pl.ds(16 * k, 16)] = accs[k]
    pltpu.sync_copy(acc_v, out_hbm.at[wid])


def _sc_partial_sums(feat, cu_pad):
    return pl.kernel(
        _sc_body,
        out_type=jax.ShapeDtypeStruct((NW, B, C), jnp.float32),
        mesh=plsc.VectorSubcoreMesh(core_axis_name="c", subcore_axis_name="s"),
        scratch_types=[
            pltpu.VMEM((CR, C), jnp.float32),
            pltpu.VMEM((CR, C), jnp.float32),
            pltpu.VMEM((32,), jnp.int32),
            pltpu.VMEM((B, C), jnp.float32),
            pltpu.SemaphoreType.DMA,
            pltpu.SemaphoreType.DMA,
        ],
    )(feat, cu_pad)


def _head_body(partials_ref, lo_ref, hi_ref, bracket_ref,
               W1_ref, b1_ref, g1_ref, be1_ref, m1_ref, v1_ref,
               W2_ref, b2_ref, g2_ref, be2_ref, m2_ref, v2_ref,
               W3_ref, b3_ref,
               pred_ref, loss_ref, cos_ref):
    sums = jnp.sum(partials_ref[...], axis=0)
    lo = lo_ref[...]
    hi = hi_ref[...]
    counts = jnp.maximum((hi - lo).astype(jnp.float32), 1.0)
    pooled = sums / counts
    h = jnp.dot(pooled, W1_ref[...], preferred_element_type=jnp.float32)
    h = h + b1_ref[...]
    h = g1_ref[...] * (h - m1_ref[...]) * lax.rsqrt(v1_ref[...] + EPS) \
        + be1_ref[...]
    h = jnp.maximum(h, 0.0)
    h = jnp.dot(h, W2_ref[...], preferred_element_type=jnp.float32)
    h = h + b2_ref[...]
    h = g2_ref[...] * (h - m2_ref[...]) * lax.rsqrt(v2_ref[...] + EPS) \
        + be2_ref[...]
    h = jnp.maximum(h, 0.0)
    pred = jnp.dot(h, W3_ref[...], preferred_element_type=jnp.float32)
    pred = pred + b3_ref[...]
    pred_ref[...] = pred
    target = bracket_ref[...]
    diff = pred - target
    loss_ref[...] = jnp.mean(diff * diff).reshape(1, 1)
    num = jnp.sum(pred * target, axis=1)
    den = (jnp.maximum(jnp.sqrt(jnp.sum(pred * pred, axis=1)), 1e-8)
           * jnp.maximum(jnp.sqrt(jnp.sum(target * target, axis=1)), 1e-8))
    cos_ref[...] = jnp.mean(num / den).reshape(1, 1)


def kernel(feat, cu_seqlens, bracket, W1, b1, g1, be1, m1, v1,
           W2, b2, g2, be2, m2, v2, W3, b3):
    cu_pad = jnp.concatenate(
        [cu_seqlens, jnp.zeros((32 - (B + 1),), jnp.int32)])
    partials = _sc_partial_sums(feat, cu_pad)

    lo = cu_seqlens[:-1].reshape(B, 1)
    hi = cu_seqlens[1:].reshape(B, 1)

    pred, loss, cos = pl.pallas_call(
        _head_body,
        out_shape=[
            jax.ShapeDtypeStruct((B, 3), jnp.float32),
            jax.ShapeDtypeStruct((1, 1), jnp.float32),
            jax.ShapeDtypeStruct((1, 1), jnp.float32),
        ],
    )(partials, lo, hi, bracket,
      W1, b1.reshape(1, 256), g1.reshape(1, 256), be1.reshape(1, 256),
      m1.reshape(1, 256), v1.reshape(1, 256),
      W2, b2.reshape(1, 128), g2.reshape(1, 128), be2.reshape(1, 128),
      m2.reshape(1, 128), v2.reshape(1, 128),
      W3, b3.reshape(1, 3))
    return (pred, loss[0, 0], cos[0, 0])


# speedup vs baseline: 2.0059x; 2.0059x over previous
import jax
import jax.numpy as jnp
from jax import lax
from jax.experimental import pallas as pl
from jax.experimental.pallas import tpu as pltpu

N = 32768
B = 16
C = 96
RB = 4096
NBLK = N // RB
EPS = 1e-5


def _body(lo_ref, hi_ref, featT_ref, bracket_ref,
          W1_ref, b1_ref, g1_ref, be1_ref, m1_ref, v1_ref,
          W2_ref, b2_ref, g2_ref, be2_ref, m2_ref, v2_ref,
          W3_ref, b3_ref,
          pred_ref, loss_ref, cos_ref, acc_ref):
    i = pl.program_id(0)

    @pl.when(i == 0)
    def _():
        acc_ref[...] = jnp.zeros_like(acc_ref)

    idx = lax.broadcasted_iota(jnp.int32, (RB, B), 0) + i * RB
    lo = lo_ref[...]
    hi = hi_ref[...]
    mask = ((idx >= lo) & (idx < hi)).astype(jnp.float32)
    acc_ref[...] += jnp.dot(featT_ref[...], mask,
                            preferred_element_type=jnp.float32,
                            precision=lax.Precision.HIGHEST)

    @pl.when(i == NBLK - 1)
    def _():
        counts = jnp.maximum((hi - lo).astype(jnp.float32), 1.0)
        pooledT = acc_ref[...] / counts
        h = lax.dot_general(pooledT, W1_ref[...],
                            dimension_numbers=(((0,), (0,)), ((), ())),
                            preferred_element_type=jnp.float32)
        h = h + b1_ref[...]
        h = g1_ref[...] * (h - m1_ref[...]) * lax.rsqrt(v1_ref[...] + EPS) \
            + be1_ref[...]
        h = jnp.maximum(h, 0.0)
        h = jnp.dot(h, W2_ref[...], preferred_element_type=jnp.float32)
        h = h + b2_ref[...]
        h = g2_ref[...] * (h - m2_ref[...]) * lax.rsqrt(v2_ref[...] + EPS) \
            + be2_ref[...]
        h = jnp.maximum(h, 0.0)
        pred = jnp.dot(h, W3_ref[...], preferred_element_type=jnp.float32)
        pred = pred + b3_ref[...]
        pred_ref[...] = pred
        target = bracket_ref[...]
        diff = pred - target
        loss_ref[...] = jnp.mean(diff * diff).reshape(1, 1)
        num = jnp.sum(pred * target, axis=1)
        den = (jnp.maximum(jnp.sqrt(jnp.sum(pred * pred, axis=1)), 1e-8)
               * jnp.maximum(jnp.sqrt(jnp.sum(target * target, axis=1)), 1e-8))
        cos_ref[...] = jnp.mean(num / den).reshape(1, 1)


def kernel(feat, cu_seqlens, bracket, W1, b1, g1, be1, m1, v1,
           W2, b2, g2, be2, m2, v2, W3, b3):
    lo = cu_seqlens[:-1].reshape(1, B)
    hi = cu_seqlens[1:].reshape(1, B)

    def whole(shape):
        return pl.BlockSpec(shape, lambda i: (0,) * len(shape))

    grid_spec = pltpu.PrefetchScalarGridSpec(
        num_scalar_prefetch=0,
        grid=(NBLK,),
        in_specs=[
            whole((1, B)),
            whole((1, B)),
            pl.BlockSpec((C, RB), lambda i: (0, i)),
            whole((B, 3)),
            whole((C, 256)), whole((1, 256)), whole((1, 256)),
            whole((1, 256)), whole((1, 256)), whole((1, 256)),
            whole((256, 128)), whole((1, 128)), whole((1, 128)),
            whole((1, 128)), whole((1, 128)), whole((1, 128)),
            whole((128, 3)), whole((1, 3)),
        ],
        out_specs=[
            whole((B, 3)),
            whole((1, 1)),
            whole((1, 1)),
        ],
        scratch_shapes=[pltpu.VMEM((C, B), jnp.float32)],
    )

    pred, loss, cos = pl.pallas_call(
        _body,
        grid_spec=grid_spec,
        out_shape=[
            jax.ShapeDtypeStruct((B, 3), jnp.float32),
            jax.ShapeDtypeStruct((1, 1), jnp.float32),
            jax.ShapeDtypeStruct((1, 1), jnp.float32),
        ],
    )(lo, hi, feat.T, bracket,
      W1, b1.reshape(1, 256), g1.reshape(1, 256), be1.reshape(1, 256),
      m1.reshape(1, 256), v1.reshape(1, 256),
      W2, b2.reshape(1, 128), g2.reshape(1, 128), be2.reshape(1, 128),
      m2.reshape(1, 128), v2.reshape(1, 128),
      W3, b3.reshape(1, 3))
    return (pred, loss[0, 0], cos[0, 0])
